# baseline (speedup 1.0000x reference)
import jax
import jax.numpy as jnp
from jax import lax
from jax.experimental import pallas as pl
from jax.experimental.pallas import tpu as pltpu
from jax.experimental.pallas import tpu_sc as plsc

NC = 2
NS = 16
NW = NC * NS

V = 100000
D = 128
G = 4
CHUNK = 128
NB = 2
N_ROWS = 4096 * 50
ROWS_PER_W = N_ROWS // NW
NCHUNK = ROWS_PER_W // CHUNK
DW = D // 2


def _body(w_hbm, s_hbm, idx_hbm, out_hbm,
          idx_v, w_buf, s_buf, o_buf, g_sems, o_sems):
    wid = lax.axis_index("s") * NC + lax.axis_index("c")

    pltpu.sync_copy(idx_hbm.at[wid], idx_v)

    lane = lax.iota(jnp.int32, 16)
    col01 = (lane >= 8).astype(jnp.int32)
    wcol_e = lane * 2
    wcol_o = lane * 2 + 1

    def start(j, b):
        idx_row = idx_v.at[j]
        dw = pltpu.async_copy(w_hbm.at[idx_row], w_buf.at[b], g_sems.at[b])
        ds = pltpu.async_copy(s_hbm.at[idx_row], s_buf.at[b], g_sems.at[b])
        return (dw, ds)

    def compute(b):
        w_ref = w_buf.at[b]
        s_ref = s_buf.at[b]
        o_ref = o_buf.at[b]

        def row_body(r, carry):
            rvec = jnp.full((16,), r, jnp.int32)
            ovec = jnp.full((16,), lax.shift_right_logical(r, 1), jnp.int32)
            obase = (r & 1) * DW
            for h in range(2):
                w8 = w_ref[r, pl.ds(h * 64, 64)]
                lo, hi = plsc.unpack(w8, format=plsc.PackFormat.INTERLEAVED,
                                     preferred_element_type=jnp.int16)
                svi = plsc.load_gather(s_ref, [rvec, col01 + 2 * h])
                sv = plsc.bitcast(svi, jnp.float16)
                plo = lo.astype(jnp.float16) * sv
                phi = hi.astype(jnp.float16) * sv
                lo32 = plsc.bitcast(plo, jnp.int32)
                hi32 = plsc.bitcast(phi, jnp.int32)
                we = (lo32 & 0xFFFF) | (hi32 << 16)
                wo = lax.shift_right_logical(lo32, 16) | (hi32 & -65536)
                plsc.store_scatter(o_ref, [ovec, obase + wcol_e + 32 * h], we)
                plsc.store_scatter(o_ref, [ovec, obase + wcol_o + 32 * h], wo)
            return carry

        lax.fori_loop(0, CHUNK, row_body, 0)

    base2 = wid * (ROWS_PER_W // 2)
    pending_g = [None] * NB
    pending_o = [None] * NB

    pending_g[0] = start(0, 0)
    for j in range(NCHUNK):
        b = j % NB
        for d in pending_g[b]:
            d.wait()
        pending_g[b] = None
        if j + 1 < NCHUNK:
            pending_g[(j + 1) % NB] = start(j + 1, (j + 1) % NB)
        if pending_o[b] is not None:
            pending_o[b].wait()
            pending_o[b] = None
        compute(b)
        pending_o[b] = pltpu.async_copy(
            o_buf.at[b],
            out_hbm.at[pl.ds(base2 + j * (CHUNK // 2), CHUNK // 2)],
            o_sems.at[b])
    for b in range(NB):
        if pending_o[b] is not None:
            pending_o[b].wait()


@jax.jit
def _run(weight, scales32, idx3):
    mesh = plsc.VectorSubcoreMesh(core_axis_name="c", subcore_axis_name="s",
                                  num_cores=NC, num_subcores=NS)
    return pl.kernel(
        _body,
        out_type=jax.ShapeDtypeStruct((N_ROWS // 2, 2 * DW), jnp.int32),
        mesh=mesh,
        scratch_types=[
            pltpu.VMEM((NCHUNK, CHUNK), jnp.int32),
            pltpu.VMEM((NB, CHUNK, D), jnp.int8),
            pltpu.VMEM((NB, CHUNK, G), jnp.int32),
            pltpu.VMEM((NB, CHUNK // 2, 2 * DW), jnp.int32),
            pltpu.SemaphoreType.DMA((NB,)),
            pltpu.SemaphoreType.DMA((NB,)),
        ],
        compiler_params=pltpu.CompilerParams(needs_layout_passes=False,
                                             use_tc_tiling_on_sc=False),
    )(weight, scales32, idx3)


def kernel(weight, scales, indices):
    B, L = indices.shape
    sbits = lax.bitcast_convert_type(scales, jnp.uint16).astype(jnp.uint32)
    sdup = lax.bitcast_convert_type(sbits * jnp.uint32(0x10001), jnp.int32)
    out_words = _run(weight, sdup, indices.reshape(NW, NCHUNK, CHUNK))
    out = lax.bitcast_convert_type(out_words, jnp.float16)
    return out.reshape(B, L, D)


# speedup vs baseline: 9.4923x; 9.4923x over previous
import jax
import jax.numpy as jnp
from jax import lax
from jax.experimental import pallas as pl
from jax.experimental.pallas import tpu as pltpu
from jax.experimental.pallas import tpu_sc as plsc

NC = 2
NS = 16
NW = NC * NS

V = 100000
D = 128
G = 4
CHUNK = 128
NB = 2
N_ROWS = 4096 * 50
ROWS_PER_W = N_ROWS // NW
NCHUNK = ROWS_PER_W // CHUNK
DW = D // 2


def _body(w_hbm, s_hbm, idx_hbm, out_hbm,
          idx_v, w_buf, s_buf, o_buf, g_sems, o_sems):
    wid = lax.axis_index("s") * NC + lax.axis_index("c")

    pltpu.sync_copy(idx_hbm.at[wid], idx_v)

    lane = lax.iota(jnp.int32, 16)
    col01 = (lane >= 8).astype(jnp.int32)
    wcol_e = lane * 2
    wcol_o = lane * 2 + 1

    def start(j, b):
        idx_row = idx_v.at[j]
        dw = pltpu.async_copy(w_hbm.at[idx_row], w_buf.at[b], g_sems.at[b])
        ds = pltpu.async_copy(s_hbm.at[idx_row], s_buf.at[b], g_sems.at[b])
        return (dw, ds)

    def compute(b):
        w_ref = w_buf.at[b]
        s_ref = s_buf.at[b]
        o_ref = o_buf.at[b]

        def row_body(r, carry):
            rvec = jnp.full((16,), r, jnp.int32)
            ovec = jnp.full((16,), r >> 1, jnp.int32)
            obase = (r & 1) * DW
            for h in range(2):
                w8 = w_ref[r, pl.ds(h * 64, 64)]
                lo, hi = plsc.unpack(w8, format=plsc.PackFormat.INTERLEAVED,
                                     preferred_element_type=jnp.int16)
                svi = plsc.load_gather(s_ref, [rvec, col01 + 2 * h])
                sv = plsc.bitcast(svi, jnp.float16)
                plo = lo.astype(jnp.float16) * sv
                phi = hi.astype(jnp.float16) * sv
                lo32 = plsc.bitcast(plo, jnp.int32)
                hi32 = plsc.bitcast(phi, jnp.int32)
                we = (lo32 & 0xFFFF) | (hi32 << 16)
                wo = lax.shift_right_logical(lo32, 16) | (hi32 & -65536)
                plsc.store_scatter(o_ref, [ovec, obase + wcol_e + 32 * h], we)
                plsc.store_scatter(o_ref, [ovec, obase + wcol_o + 32 * h], wo)
            return carry

        lax.fori_loop(0, CHUNK, row_body, 0)

    base2 = wid * (ROWS_PER_W // 2)
    pending_g = [None] * NB
    pending_o = [None] * NB

    pending_g[0] = start(0, 0)
    for j in range(NCHUNK):
        b = j % NB
        for d in pending_g[b]:
            d.wait()
        pending_g[b] = None
        if j + 1 < NCHUNK:
            pending_g[(j + 1) % NB] = start(j + 1, (j + 1) % NB)
        if pending_o[b] is not None:
            pending_o[b].wait()
            pending_o[b] = None
        compute(b)
        pending_o[b] = pltpu.async_copy(
            o_buf.at[b],
            out_hbm.at[pl.ds(base2 + j * (CHUNK // 2), CHUNK // 2)],
            o_sems.at[b])
    for b in range(NB):
        if pending_o[b] is not None:
            pending_o[b].wait()


@jax.jit
def _run(weight, scales32, idx3):
    mesh = plsc.VectorSubcoreMesh(core_axis_name="c", subcore_axis_name="s",
                                  num_cores=NC, num_subcores=NS)
    return pl.kernel(
        _body,
        out_type=jax.ShapeDtypeStruct((N_ROWS // 2, 2 * DW), jnp.int32),
        mesh=mesh,
        scratch_types=[
            pltpu.VMEM((NCHUNK, CHUNK), jnp.int32),
            pltpu.VMEM((NB, CHUNK, D), jnp.int8),
            pltpu.VMEM((NB, CHUNK, G), jnp.int32),
            pltpu.VMEM((NB, CHUNK // 2, 2 * DW), jnp.int32),
            pltpu.SemaphoreType.DMA((NB,)),
            pltpu.SemaphoreType.DMA((NB,)),
        ],
        compiler_params=pltpu.CompilerParams(needs_layout_passes=False,
                                             use_tc_tiling_on_sc=False),
    )(weight, scales32, idx3)


def kernel(weight, scales, indices):
    B, L = indices.shape
    sbits = lax.bitcast_convert_type(scales, jnp.uint16).astype(jnp.uint32)
    sdup = lax.bitcast_convert_type(sbits * jnp.uint32(0x10001), jnp.int32)
    out_words = _run(weight, sdup, indices.reshape(NW, NCHUNK, CHUNK))
    out = lax.bitcast_convert_type(out_words.reshape(N_ROWS, DW), jnp.float16)
    return out.reshape(B, L, D)


# speedup vs baseline: 9.5246x; 1.0034x over previous
import jax
import jax.numpy as jnp
from jax import lax
from jax.experimental import pallas as pl
from jax.experimental.pallas import tpu as pltpu
from jax.experimental.pallas import tpu_sc as plsc

NC = 2
NS = 16
NW = NC * NS

V = 100000
D = 128
G = 4
CHUNK = 50
NB = 2
N_ROWS = 4096 * 50
ROWS_PER_W = N_ROWS // NW
NCHUNK = ROWS_PER_W // CHUNK
DW = D // 2
OROW = CHUNK // 2


def _body(w_hbm, s_hbm, idx_hbm, out_hbm,
          idx_v, w_buf, s_buf, o_buf, g_sems, o_sems):
    wid = lax.axis_index("s") * NC + lax.axis_index("c")

    pltpu.sync_copy(idx_hbm.at[pl.ds(wid * NCHUNK, NCHUNK)], idx_v)

    lane = lax.iota(jnp.int32, 16)
    col01 = (lane >= 8).astype(jnp.int32)
    wcol_e = lane * 2
    wcol_o = lane * 2 + 1

    base2 = wid * (ROWS_PER_W // 2)

    def gather(j, b):
        idx_row = idx_v.at[j]
        dw = pltpu.make_async_copy(w_hbm.at[idx_row], w_buf.at[b],
                                   g_sems.at[b])
        ds = pltpu.make_async_copy(s_hbm.at[idx_row], s_buf.at[b],
                                   g_sems.at[b])
        return (dw, ds)

    def writeback(j, b):
        return pltpu.make_async_copy(
            o_buf.at[b], out_hbm.at[pl.ds(base2 + j * OROW, OROW)],
            o_sems.at[b])

    def compute(b):
        w_ref = w_buf.at[b]
        s_ref = s_buf.at[b]
        o_ref = o_buf.at[b]

        def row_body(r, carry):
            rvec = jnp.full((16,), r, jnp.int32)
            ovec = jnp.full((16,), r >> 1, jnp.int32)
            obase = (r & 1) * DW
            for h in range(2):
                w8 = w_ref[r, pl.ds(h * 64, 64)]
                lo, hi = plsc.unpack(w8, format=plsc.PackFormat.INTERLEAVED,
                                     preferred_element_type=jnp.int16)
                svi = plsc.load_gather(s_ref, [rvec, col01 + 2 * h])
                sv = plsc.bitcast(svi, jnp.float16)
                plo = lo.astype(jnp.float16) * sv
                phi = hi.astype(jnp.float16) * sv
                lo32 = plsc.bitcast(plo, jnp.int32)
                hi32 = plsc.bitcast(phi, jnp.int32)
                we = (lo32 & 0xFFFF) | (hi32 << 16)
                wo = lax.shift_right_logical(lo32, 16) | (hi32 & -65536)
                plsc.store_scatter(o_ref, [ovec, obase + wcol_e + 32 * h], we)
                plsc.store_scatter(o_ref, [ovec, obase + wcol_o + 32 * h], wo)
            return carry

        lax.fori_loop(0, CHUNK, row_body, 0)

    for d in gather(0, 0):
        d.start()
    for b in range(NB):
        writeback(b, b).start()

    def chunk_body(j, carry):
        b = j & 1
        nb = 1 - b
        jn = jnp.minimum(j + 1, NCHUNK - 1)
        for d in gather(jn, nb):
            d.start()
        for d in gather(j, b):
            d.wait()
        writeback(j, b).wait()
        compute(b)
        writeback(j, b).start()
        return carry

    lax.fori_loop(0, NCHUNK, chunk_body, 0)

    for d in gather(NCHUNK - 1, NCHUNK & 1):
        d.wait()
    for b in range(NB):
        writeback(NCHUNK - NB + b, (NCHUNK - NB + b) & 1).wait()


@jax.jit
def _run(weight, scales32, indices):
    mesh = plsc.VectorSubcoreMesh(core_axis_name="c", subcore_axis_name="s",
                                  num_cores=NC, num_subcores=NS)
    return pl.kernel(
        _body,
        out_type=jax.ShapeDtypeStruct((N_ROWS // 2, 2 * DW), jnp.int32),
        mesh=mesh,
        scratch_types=[
            pltpu.VMEM((NCHUNK, CHUNK), jnp.int32),
            pltpu.VMEM((NB, CHUNK, D), jnp.int8),
            pltpu.VMEM((NB, CHUNK, G), jnp.int32),
            pltpu.VMEM((NB, OROW, 2 * DW), jnp.int32),
            pltpu.SemaphoreType.DMA((NB,)),
            pltpu.SemaphoreType.DMA((NB,)),
        ],
        compiler_params=pltpu.CompilerParams(needs_layout_passes=False,
                                             use_tc_tiling_on_sc=False),
    )(weight, scales32, indices)


def kernel(weight, scales, indices):
    B, L = indices.shape
    sbits = lax.bitcast_convert_type(scales, jnp.uint16).astype(jnp.uint32)
    sdup = lax.bitcast_convert_type(sbits * jnp.uint32(0x10001), jnp.int32)
    out_words = _run(weight, sdup, indices)
    out = lax.bitcast_convert_type(out_words.reshape(N_ROWS, DW), jnp.float16)
    return out.reshape(B, L, D)


# speedup vs baseline: 16.7153x; 1.7550x over previous
import jax
import jax.numpy as jnp
from jax import lax
from jax.experimental import pallas as pl
from jax.experimental.pallas import tpu as pltpu
from jax.experimental.pallas import tpu_sc as plsc

NC = 2
NS = 16
NW = NC * NS

V = 100000
D = 128
G = 4
CHUNK = 50
NB = 2
N_ROWS = 4096 * 50
ROWS_PER_W = N_ROWS // NW
NCHUNK = ROWS_PER_W // CHUNK
DW = D // 2
OROW = CHUNK // 2


def _body(w_hbm, s_hbm, idx_hbm, out_hbm,
          idx_v, w_buf, s_buf, o_buf, g_sems, o_sems):
    wid = lax.axis_index("s") * NC + lax.axis_index("c")

    pltpu.sync_copy(idx_hbm.at[pl.ds(wid * NCHUNK, NCHUNK)], idx_v)

    lane = lax.iota(jnp.int32, 16)
    col01 = (lane >= 8).astype(jnp.int32)
    wcol_e = lane * 2
    wcol_o = lane * 2 + 1

    def gather(j, b):
        idx_row = idx_v.at[j]
        dw = pltpu.make_async_copy(w_hbm.at[idx_row], w_buf.at[b],
                                   g_sems.at[b])
        ds = pltpu.make_async_copy(s_hbm.at[idx_row], s_buf.at[b],
                                   g_sems.at[b])
        return (dw, ds)

    def writeback(j, b):
        return pltpu.make_async_copy(
            o_buf.at[b],
            out_hbm.at[pl.ds((wid * NCHUNK + j) * 32, OROW)],
            o_sems.at[b])

    def compute(b):
        w_ref = w_buf.at[b]
        s_ref = s_buf.at[b]
        o_ref = o_buf.at[b]

        def row_body(r, carry):
            rvec = jnp.full((16,), r, jnp.int32)
            ovec = jnp.full((16,), r >> 1, jnp.int32)
            obase = (r & 1) * DW
            for h in range(2):
                w8 = w_ref[r, pl.ds(h * 64, 64)]
                lo, hi = plsc.unpack(w8, format=plsc.PackFormat.INTERLEAVED,
                                     preferred_element_type=jnp.int16)
                svi = plsc.load_gather(s_ref, [rvec, col01 + 2 * h])
                sv = plsc.bitcast(svi, jnp.float16)
                plo = lo.astype(jnp.float16) * sv
                phi = hi.astype(jnp.float16) * sv
                lo32 = plsc.bitcast(plo, jnp.int32)
                hi32 = plsc.bitcast(phi, jnp.int32)
                we = (lo32 & 0xFFFF) | (hi32 << 16)
                wo = lax.shift_right_logical(lo32, 16) | (hi32 & -65536)
                plsc.store_scatter(o_ref, [ovec, obase + wcol_e + 32 * h], we)
                plsc.store_scatter(o_ref, [ovec, obase + wcol_o + 32 * h], wo)
            return carry

        lax.fori_loop(0, CHUNK, row_body, 0)

    for d in gather(0, 0):
        d.start()
    for b in range(NB):
        writeback(b, b).start()

    def chunk_body(j, carry):
        b = j & 1
        nb = 1 - b
        jn = jnp.minimum(j + 1, NCHUNK - 1)
        for d in gather(jn, nb):
            d.start()
        for d in gather(j, b):
            d.wait()
        writeback(j, b).wait()
        compute(b)
        writeback(j, b).start()
        return carry

    lax.fori_loop(0, NCHUNK, chunk_body, 0)

    for d in gather(NCHUNK - 1, NCHUNK & 1):
        d.wait()
    for b in range(NB):
        writeback(NCHUNK - NB + b, (NCHUNK - NB + b) & 1).wait()


@jax.jit
def _run(weight, scales32, indices):
    mesh = plsc.VectorSubcoreMesh(core_axis_name="c", subcore_axis_name="s",
                                  num_cores=NC, num_subcores=NS)
    return pl.kernel(
        _body,
        out_type=jax.ShapeDtypeStruct((4096 * 32, 2 * DW), jnp.int32),
        mesh=mesh,
        scratch_types=[
            pltpu.VMEM((NCHUNK, CHUNK), jnp.int32),
            pltpu.VMEM((NB, CHUNK, D), jnp.int8),
            pltpu.VMEM((NB, CHUNK, G), jnp.int32),
            pltpu.VMEM((NB, OROW, 2 * DW), jnp.int32),
            pltpu.SemaphoreType.DMA((NB,)),
            pltpu.SemaphoreType.DMA((NB,)),
        ],
        compiler_params=pltpu.CompilerParams(needs_layout_passes=False,
                                             use_tc_tiling_on_sc=False),
    )(weight, scales32, indices)


def kernel(weight, scales, indices):
    B, L = indices.shape
    sbits = lax.bitcast_convert_type(scales, jnp.uint16).astype(jnp.uint32)
    sdup = lax.bitcast_convert_type(sbits * jnp.uint32(0x10001), jnp.int32)
    out_words = _run(weight, sdup, indices)
    out = lax.bitcast_convert_type(out_words.reshape(B, 32, D), jnp.float16)
    return out.reshape(B, 64, D)[:, :L, :]
